# baseline (speedup 1.0000x reference)
import jax
import jax.numpy as jnp
from jax.experimental import pallas as pl
from jax.experimental.pallas import tpu as pltpu

_B, _C, _H, _W = 16, 256, 32, 32


_SPLIT = 1


def _body(row_ref, col_ref, out_ref, scratch, sems):
    scratch[:, :, :_C] = jnp.broadcast_to(col_ref[...][None, :, :], (_H, _W, _C))
    scratch[:, :, _C:] = jnp.broadcast_to(row_ref[...][:, None, :], (_H, _W, _C))
    hh = _H // _SPLIT
    for b in range(_B):
        for s in range(_SPLIT):
            pltpu.make_async_copy(
                scratch.at[pl.ds(s * hh, hh)],
                out_ref.at[b, pl.ds(s * hh, hh)],
                sems.at[b * _SPLIT + s]).start(priority=1)
    for b in range(_B):
        for s in range(_SPLIT):
            pltpu.make_async_copy(
                scratch.at[pl.ds(s * hh, hh)],
                out_ref.at[b, pl.ds(s * hh, hh)],
                sems.at[b * _SPLIT + s]).wait()


def kernel(mask, row_embed, col_embed):
    b = mask.shape[0]
    h, w = mask.shape[-2], mask.shape[-1]
    c = row_embed.shape[-1]
    out = pl.pallas_call(
        _body,
        grid=(1,),
        in_specs=[
            pl.BlockSpec((h, c), lambda i: (0, 0)),
            pl.BlockSpec((w, c), lambda i: (0, 0)),
        ],
        out_specs=pl.BlockSpec(memory_space=pl.ANY),
        out_shape=jax.ShapeDtypeStruct((b, h, w, 2 * c), jnp.float32),
        scratch_shapes=[
            pltpu.VMEM((h, w, 2 * c), jnp.float32),
            pltpu.SemaphoreType.DMA((b * _SPLIT,)),
        ],
    )(row_embed, col_embed)
    return jnp.transpose(out, (0, 3, 1, 2))


# speedup vs baseline: 1.0049x; 1.0049x over previous
import jax
import jax.numpy as jnp
from jax.experimental import pallas as pl
from jax.experimental.pallas import tpu as pltpu

_B, _C, _H, _W = 16, 256, 32, 32


def _body(row_ref, col_ref, out_ref, scratch, sems):
    scratch[:, :, :_C] = jnp.broadcast_to(col_ref[...][None, :, :], (_H, _W, _C))
    scratch[:, :, _C:] = jnp.broadcast_to(row_ref[...][:, None, :], (_H, _W, _C))
    for b in range(_B):
        pltpu.make_async_copy(scratch, out_ref.at[b], sems.at[b]).start()
    for b in range(_B):
        pltpu.make_async_copy(scratch, out_ref.at[b], sems.at[b]).wait()


def kernel(mask, row_embed, col_embed):
    b = mask.shape[0]
    h, w = mask.shape[-2], mask.shape[-1]
    c = row_embed.shape[-1]
    out = pl.pallas_call(
        _body,
        grid=(1,),
        in_specs=[
            pl.BlockSpec((h, c), lambda i: (0, 0)),
            pl.BlockSpec((w, c), lambda i: (0, 0)),
        ],
        out_specs=pl.BlockSpec(memory_space=pl.ANY),
        out_shape=jax.ShapeDtypeStruct((b, h, w, 2 * c), jnp.float32),
        scratch_shapes=[
            pltpu.VMEM((h, w, 2 * c), jnp.float32),
            pltpu.SemaphoreType.DMA((b,)),
        ],
    )(row_embed, col_embed)
    return jnp.transpose(out, (0, 3, 1, 2))
